# Initial kernel scaffold; baseline (speedup 1.0000x reference)
import jax
import jax.numpy as jnp
from jax.experimental import pallas as pl


def kernel(recon_signal_error, original_idx, params):
    raise NotImplementedError("write your pallas kernel here")


# speedup vs baseline: 7.1130x; 7.1130x over previous
import functools

import jax
import jax.numpy as jnp
from jax import lax
from jax.experimental import pallas as pl
from jax.experimental.pallas import tpu as pltpu

BD = 256
BS = 1024


def _leaky(x, slope):
    return jnp.where(x >= 0, x, slope * x)


def _k1_body(h_ref, w_ref, avm_ref, hp_ref, av_ref, gmax_ref):
    i = pl.program_id(0)
    hp = jnp.dot(h_ref[...], w_ref[...], preferred_element_type=jnp.float32)
    hp_ref[...] = hp
    av = jnp.dot(hp, avm_ref[...], preferred_element_type=jnp.float32)
    av_ref[...] = av
    bmax = jnp.full((1, 128), jnp.max(av[:, 0]), jnp.float32)

    @pl.when(i == 0)
    def _():
        gmax_ref[...] = bmax

    @pl.when(i > 0)
    def _():
        gmax_ref[...] = jnp.maximum(gmax_ref[...], bmax)


def _k1(h, W, asrc, adst):
    npad, din = h.shape
    dout = W.shape[1]
    avm = jnp.pad(jnp.stack([asrc, adst], axis=1), ((0, 0), (0, 126)))
    grid = (npad // BD,)
    hp, av, gmax = pl.pallas_call(
        _k1_body,
        grid=grid,
        in_specs=[
            pl.BlockSpec((BD, din), lambda i: (i, 0)),
            pl.BlockSpec((din, dout), lambda i: (0, 0)),
            pl.BlockSpec((dout, 128), lambda i: (0, 0)),
        ],
        out_specs=[
            pl.BlockSpec((BD, dout), lambda i: (i, 0)),
            pl.BlockSpec((BD, 128), lambda i: (i, 0)),
            pl.BlockSpec((1, 128), lambda i: (0, 0)),
        ],
        out_shape=[
            jax.ShapeDtypeStruct((npad, dout), jnp.float32),
            jax.ShapeDtypeStruct((npad, 128), jnp.float32),
            jax.ShapeDtypeStruct((1, 128), jnp.float32),
        ],
        compiler_params=pltpu.CompilerParams(
            dimension_semantics=("arbitrary",)),
    )(h, W, avm)
    return hp, av, gmax


def _k3_body(gmax_ref, as_ref, ad_ref, cnt_ref, hp_ref, b_ref,
             out_ref, acc_ref, den_ref, *, relu):
    k = pl.program_id(1)

    @pl.when(k == 0)
    def _():
        acc_ref[...] = jnp.zeros_like(acc_ref)
        den_ref[...] = jnp.zeros_like(den_ref)

    a_s = as_ref[0, :]
    a_d = ad_ref[:, 0]
    g = gmax_ref[0, 0]
    t = _leaky(a_s[None, :] + a_d[:, None], 0.2)
    m = _leaky(g + a_d, 0.2)
    arg = jnp.minimum(t - m[:, None], 0.0)
    wt = cnt_ref[...] * jnp.exp(arg)
    acc_ref[...] += jnp.dot(wt, hp_ref[...], preferred_element_type=jnp.float32)
    den_ref[...] += jnp.sum(wt, axis=1, keepdims=True)

    @pl.when(k == pl.num_programs(1) - 1)
    def _():
        o = acc_ref[...] / (den_ref[...] + 1e-16) + b_ref[0, :][None, :]
        if relu:
            o = jnp.maximum(o, 0.0)
        out_ref[...] = o


def _k3(gmax, a_s_row, a_d_col, cnt, hp, b, relu):
    npad, dout = hp.shape
    grid = (npad // BD, npad // BS)
    return pl.pallas_call(
        functools.partial(_k3_body, relu=relu),
        grid=grid,
        in_specs=[
            pl.BlockSpec((1, 128), lambda i, k: (0, 0)),
            pl.BlockSpec((1, BS), lambda i, k: (0, k)),
            pl.BlockSpec((BD, 1), lambda i, k: (i, 0)),
            pl.BlockSpec((BD, BS), lambda i, k: (i, k)),
            pl.BlockSpec((BS, dout), lambda i, k: (k, 0)),
            pl.BlockSpec((1, dout), lambda i, k: (0, 0)),
        ],
        out_specs=pl.BlockSpec((BD, dout), lambda i, k: (i, 0)),
        out_shape=jax.ShapeDtypeStruct((npad, dout), jnp.float32),
        scratch_shapes=[
            pltpu.VMEM((BD, dout), jnp.float32),
            pltpu.VMEM((BD, 1), jnp.float32),
        ],
        compiler_params=pltpu.CompilerParams(
            dimension_semantics=("parallel", "arbitrary")),
    )(gmax, a_s_row, a_d_col, cnt, hp, b)


def _k4_body(cnt_ref, g_ref, w1_ref, b1_ref, w2_ref, b2_ref,
             *rest, act, cls):
    if cls:
        cw1_ref, cb1_ref, cw2_ref, cb2_ref, out_ref, acc_ref = rest
    else:
        out_ref, acc_ref = rest
    k = pl.program_id(1)

    @pl.when(k == 0)
    def _():
        acc_ref[...] = jnp.zeros_like(acc_ref)

    acc_ref[...] += jnp.dot(cnt_ref[...], g_ref[...],
                            preferred_element_type=jnp.float32)

    @pl.when(k == pl.num_programs(1) - 1)
    def _():
        z = acc_ref[...]
        z = jnp.maximum(jnp.dot(z, w1_ref[...],
                                preferred_element_type=jnp.float32)
                        + b1_ref[0, :][None, :], 0.0)
        z = jnp.dot(z, w2_ref[...],
                    preferred_element_type=jnp.float32) + b2_ref[0, :][None, :]
        if act:
            z = jnp.maximum(z, 0.0)
        if cls:
            z = _leaky(jnp.dot(z, cw1_ref[...],
                               preferred_element_type=jnp.float32)
                       + cb1_ref[0, :][None, :], 0.01)
            z = jnp.dot(z, cw2_ref[...],
                        preferred_element_type=jnp.float32) \
                + cb2_ref[0, :][None, :]
        out_ref[...] = z


def _k4(cnt, g, w1, b1, w2, b2, act, cls):
    npad, din = g.shape
    dout = 1 if cls is not None else w2.shape[1]
    grid = (npad // BD, npad // BS)
    dh = w1.shape[1]
    d2 = w2.shape[1]
    in_specs = [
        pl.BlockSpec((BD, BS), lambda i, k: (i, k)),
        pl.BlockSpec((BS, din), lambda i, k: (k, 0)),
        pl.BlockSpec((din, dh), lambda i, k: (0, 0)),
        pl.BlockSpec((1, dh), lambda i, k: (0, 0)),
        pl.BlockSpec((dh, d2), lambda i, k: (0, 0)),
        pl.BlockSpec((1, d2), lambda i, k: (0, 0)),
    ]
    args = [cnt, g, w1, b1.reshape(1, -1), w2, b2.reshape(1, -1)]
    if cls is not None:
        cw1, cb1, cw2, cb2 = cls
        in_specs += [
            pl.BlockSpec((d2, 10), lambda i, k: (0, 0)),
            pl.BlockSpec((1, 10), lambda i, k: (0, 0)),
            pl.BlockSpec((10, 1), lambda i, k: (0, 0)),
            pl.BlockSpec((1, 1), lambda i, k: (0, 0)),
        ]
        args += [cw1, cb1.reshape(1, -1), cw2, cb2.reshape(1, -1)]
    return pl.pallas_call(
        functools.partial(_k4_body, act=act, cls=cls is not None),
        grid=grid,
        in_specs=in_specs,
        out_specs=pl.BlockSpec((BD, dout), lambda i, k: (i, 0)),
        out_shape=jax.ShapeDtypeStruct((npad, dout), jnp.float32),
        scratch_shapes=[pltpu.VMEM((BD, din), jnp.float32)],
        compiler_params=pltpu.CompilerParams(
            dimension_semantics=("parallel", "arbitrary")),
    )(*args)


def _build_cnt(flat_idx, npad):
    cnt = jnp.zeros((npad * npad,), jnp.float32).at[flat_idx].add(1.0)
    return cnt.reshape(npad, npad)


def kernel(recon_signal_error, original_idx, params):
    x = recon_signal_error
    n, c = x.shape
    npad = (n // 2048 + 1) * 2048

    src0 = original_idx[0]
    dst0 = original_idx[1]
    loop = jnp.arange(n, dtype=jnp.int32)
    src = jnp.concatenate([src0, loop])
    dst = jnp.concatenate([dst0, loop])
    flat = dst * npad + src

    cnt = _build_cnt(flat, npad)

    h = jnp.pad(x, ((0, npad - n), (0, 0)))
    for l in range(4):
        hp, av, gmax = _k1(h, params['gat%d_W' % l],
                           params['gat%d_asrc' % l], params['gat%d_adst' % l])
        a_s_row = av[:, 0].reshape(1, npad)
        a_d_col = av[:, 1:2]
        h = _k3(gmax, a_s_row, a_d_col, cnt, hp,
                params['gat%d_b' % l].reshape(1, -1), relu=(l < 3))

    g = _k4(cnt, h, params['gin0_W1'], params['gin0_b1'],
            params['gin0_W2'], params['gin0_b2'], act=True, cls=None)
    out = _k4(cnt, g, params['gin1_W1'], params['gin1_b1'],
              params['gin1_W2'], params['gin1_b2'], act=False,
              cls=(params['cls_W1'], params['cls_b1'],
                   params['cls_W2'], params['cls_b2']))
    return out[:n]
